# baseline (speedup 1.0000x reference)
import functools

import jax
import jax.numpy as jnp
from jax import lax
from jax.experimental import pallas as pl
from jax.experimental.pallas import tpu as pltpu
from jax.experimental.pallas import tpu_sc as plsc

B = 4096
D = 32

_info = plsc.get_sparse_core_info()
_NC, _NS = _info.num_cores, _info.num_subcores
_NW = _NC * _NS
_BPW = B // _NW

_mesh = plsc.VectorSubcoreMesh(core_axis_name="c", subcore_axis_name="s")


@functools.partial(
    pl.kernel,
    mesh=_mesh,
    compiler_params=pltpu.CompilerParams(use_tc_tiling_on_sc=False),
    out_type=[
        jax.ShapeDtypeStruct((B, D), jnp.float32),
        jax.ShapeDtypeStruct((B, D), jnp.float32),
    ],
    scratch_types=[
        pltpu.VMEM((_BPW,), jnp.int32),
        pltpu.VMEM((_BPW, D), jnp.float32),
        pltpu.VMEM((_BPW,), jnp.int32),
        pltpu.VMEM((_BPW, D), jnp.float32),
        pltpu.SemaphoreType.DMA,
        pltpu.SemaphoreType.DMA,
    ],
)
def _sc_gather(uidx_hbm, tidx_hbm, u_hbm, v_hbm, uout_hbm, tout_hbm,
               uidx_v, urows_v, tidx_v, trows_v, usem, tsem):
    wid = lax.axis_index("s") * _NC + lax.axis_index("c")
    base = wid * _BPW
    pltpu.sync_copy(uidx_hbm.at[pl.ds(base, _BPW)], uidx_v)
    pltpu.sync_copy(tidx_hbm.at[pl.ds(base, _BPW)], tidx_v)
    ucp = pltpu.async_copy(u_hbm.at[uidx_v], urows_v, usem)
    tcp = pltpu.async_copy(v_hbm.at[tidx_v], trows_v, tsem)
    ucp.wait()
    tcp.wait()
    pltpu.sync_copy(urows_v, uout_hbm.at[pl.ds(base, _BPW)])
    pltpu.sync_copy(trows_v, tout_hbm.at[pl.ds(base, _BPW)])


def _mm_body(a_ref, b_ref, o_ref):
    o_ref[...] = lax.dot_general(
        a_ref[...], b_ref[...],
        dimension_numbers=(((1,), (1,)), ((), ())),
        preferred_element_type=jnp.float32,
    )


_BM = 512


def _tc_matmul(u_ebd, t_ebd):
    return pl.pallas_call(
        _mm_body,
        grid=(B // _BM,),
        in_specs=[
            pl.BlockSpec((_BM, D), lambda i: (i, 0)),
            pl.BlockSpec((B, D), lambda i: (0, 0)),
        ],
        out_specs=pl.BlockSpec((_BM, B), lambda i: (i, 0)),
        out_shape=jax.ShapeDtypeStruct((B, B), jnp.float32),
    )(u_ebd, t_ebd)


def kernel(user_indices, item_seq_indices, target_item_indices, target_domain, U, V):
    uidx = user_indices.astype(jnp.int32)
    tidx = target_item_indices.reshape(B).astype(jnp.int32)
    u_ebd, t_ebd = _sc_gather(uidx, tidx, U, V)
    return _tc_matmul(u_ebd, t_ebd)


# speedup vs baseline: 9.3987x; 9.3987x over previous
import functools

import jax
import jax.numpy as jnp
from jax import lax
from jax.experimental import pallas as pl
from jax.experimental.pallas import tpu as pltpu
from jax.experimental.pallas import tpu_sc as plsc

B = 4096
D = 32
LANES = 16

_info = plsc.get_sparse_core_info()
_NC, _NS = _info.num_cores, _info.num_subcores
_NW = _NC * _NS
_BPW = B // _NW
_NB = 8
_NG = _BPW // _NB

_mesh = plsc.VectorSubcoreMesh(core_axis_name="c", subcore_axis_name="s")


@functools.partial(
    pl.kernel,
    mesh=_mesh,
    compiler_params=pltpu.CompilerParams(needs_layout_passes=False),
    out_type=[
        jax.ShapeDtypeStruct((D, B), jnp.float32),
        jax.ShapeDtypeStruct((D, B), jnp.float32),
    ],
    scratch_types=[
        pltpu.VMEM((_BPW + LANES,), jnp.int32),
        pltpu.VMEM((_BPW + LANES,), jnp.int32),
        pltpu.VMEM((_NB, D, 128), jnp.float32),
        pltpu.VMEM((_NB, D, 128), jnp.float32),
        pltpu.VMEM((D, _BPW), jnp.float32),
        pltpu.VMEM((D, _BPW), jnp.float32),
        [pltpu.SemaphoreType.DMA] * _NB,
        [pltpu.SemaphoreType.DMA] * _NB,
    ],
)
def _sc_gather(uidx_hbm, tidx_hbm, ut_hbm, vt_hbm, uout_hbm, tout_hbm,
               uidx_v, tidx_v, ubuf, tbuf, uoutT, toutT, usems, tsems):
    wid = lax.axis_index("s") * _NC + lax.axis_index("c")
    base = wid * _BPW
    pltpu.sync_copy(uidx_hbm.at[pl.ds(base, _BPW)], uidx_v.at[pl.ds(0, _BPW)])
    pltpu.sync_copy(tidx_hbm.at[pl.ds(base, _BPW)], tidx_v.at[pl.ds(0, _BPW)])

    def _fetch(ju, jt, b):
        cu = pl.multiple_of(jnp.bitwise_and(ju, -128), 128)
        ct = pl.multiple_of(jnp.bitwise_and(jt, -128), 128)
        pltpu.async_copy(ut_hbm.at[:, pl.ds(cu, 128)], ubuf.at[b], usems[b])
        pltpu.async_copy(vt_hbm.at[:, pl.ds(ct, 128)], tbuf.at[b], tsems[b])

    def _drain(b):
        pltpu.make_async_copy(ut_hbm.at[:, pl.ds(0, 128)], ubuf.at[b], usems[b]).wait()
        pltpu.make_async_copy(vt_hbm.at[:, pl.ds(0, 128)], tbuf.at[b], tsems[b]).wait()

    iota_lo = lax.iota(jnp.int32, LANES)
    iota_hi = iota_lo + LANES

    def _select(buf, outT, lane, k):
        lane_v = jnp.full((LANES,), lane, jnp.int32)
        k_v = jnp.full((LANES,), k, jnp.int32)
        lo = plsc.load_gather(buf, [iota_lo, lane_v])
        hi = plsc.load_gather(buf, [iota_hi, lane_v])
        plsc.store_scatter(outT, [iota_lo, k_v], lo)
        plsc.store_scatter(outT, [iota_hi, k_v], hi)

    uvec0 = uidx_v[pl.ds(0, LANES)]
    tvec0 = tidx_v[pl.ds(0, LANES)]
    for b in range(_NB):
        _fetch(uvec0[b], tvec0[b], b)

    def group(g, carry):
        uvec = uidx_v[pl.ds(g * _NB, LANES)]
        tvec = tidx_v[pl.ds(g * _NB, LANES)]
        for b in range(_NB):
            k = g * _NB + b
            _drain(b)
            _select(ubuf.at[b], uoutT, jnp.bitwise_and(uvec[b], 127), k)
            _select(tbuf.at[b], toutT, jnp.bitwise_and(tvec[b], 127), k)

            @pl.when(k + _NB < _BPW)
            def _():
                _fetch(uvec[b + _NB], tvec[b + _NB], b)

        return carry

    lax.fori_loop(0, _NG, group, 0)
    pltpu.sync_copy(uoutT, uout_hbm.at[:, pl.ds(base, _BPW)])
    pltpu.sync_copy(toutT, tout_hbm.at[:, pl.ds(base, _BPW)])


def _mm_body(a_ref, b_ref, o_ref):
    o_ref[...] = lax.dot_general(
        a_ref[...], b_ref[...],
        dimension_numbers=(((0,), (0,)), ((), ())),
        preferred_element_type=jnp.float32,
    )


_BM = 512


def _tc_matmul(u_ebd_t, t_ebd_t):
    return pl.pallas_call(
        _mm_body,
        grid=(B // _BM,),
        in_specs=[
            pl.BlockSpec((D, _BM), lambda i: (0, i)),
            pl.BlockSpec((D, B), lambda i: (0, 0)),
        ],
        out_specs=pl.BlockSpec((_BM, B), lambda i: (i, 0)),
        out_shape=jax.ShapeDtypeStruct((B, B), jnp.float32),
    )(u_ebd_t, t_ebd_t)


def kernel(user_indices, item_seq_indices, target_item_indices, target_domain, U, V):
    uidx = user_indices.astype(jnp.int32)
    tidx = target_item_indices.reshape(B).astype(jnp.int32)
    u_ebd_t, t_ebd_t = _sc_gather(uidx, tidx, U.T, V.T)
    return _tc_matmul(u_ebd_t, t_ebd_t)
